# baseline (speedup 1.0000x reference)
import functools

import jax
import jax.numpy as jnp
from jax import lax
from jax.experimental import pallas as pl
from jax.experimental.pallas import tpu as pltpu
from jax.experimental.pallas import tpu_sc as plsc

HID = 128
EPS = 1e-12

_SC_INFO = plsc.get_sparse_core_info()
_NC = _SC_INFO.num_cores
_NS = _SC_INFO.num_subcores
_NW = _NC * _NS

_CHUNK = 128


def _gather_body(ids_hbm, table_hbm, out_hbm,
                 idx0, idx1, rows0, rows1, gsem0, gsem1, wsem):
    wid = lax.axis_index("s") * _NC + lax.axis_index("c")
    base = wid * (2 * _CHUNK)
    pltpu.sync_copy(ids_hbm.at[pl.ds(base, _CHUNK)], idx0)
    g0 = pltpu.async_copy(table_hbm.at[idx0], rows0, gsem0)
    pltpu.sync_copy(ids_hbm.at[pl.ds(base + _CHUNK, _CHUNK)], idx1)
    g1 = pltpu.async_copy(table_hbm.at[idx1], rows1, gsem1)
    g0.wait()
    w0 = pltpu.async_copy(rows0, out_hbm.at[pl.ds(base, _CHUNK)], wsem)
    g1.wait()
    w1 = pltpu.async_copy(rows1, out_hbm.at[pl.ds(base + _CHUNK, _CHUNK)],
                          wsem)
    w0.wait()
    w1.wait()


def _sc_gather(ids_flat, word_emb):
    n_tokens = ids_flat.shape[0]
    assert n_tokens == _NW * 2 * _CHUNK
    mesh = plsc.VectorSubcoreMesh(core_axis_name="c", subcore_axis_name="s")
    k = functools.partial(
        pl.kernel,
        mesh=mesh,
        out_type=jax.ShapeDtypeStruct((n_tokens, HID), jnp.float32),
        scratch_types=[
            pltpu.VMEM((_CHUNK,), jnp.int32),
            pltpu.VMEM((_CHUNK,), jnp.int32),
            pltpu.VMEM((_CHUNK, HID), jnp.float32),
            pltpu.VMEM((_CHUNK, HID), jnp.float32),
            pltpu.SemaphoreType.DMA,
            pltpu.SemaphoreType.DMA,
            pltpu.SemaphoreType.DMA,
        ],
    )(_gather_body)
    return k(ids_flat, word_emb)


def _tc_body(x_ref, pos_ref, type_ref, gamma_ref, beta_ref, w_ref, b_ref,
             out_ref):
    x = x_ref[...] + pos_ref[...] + type_ref[...]
    mean = jnp.mean(x, axis=-1, keepdims=True)
    xc = x - mean
    var = jnp.mean(xc * xc, axis=-1, keepdims=True)
    normed = xc * lax.rsqrt(var + EPS)
    y = normed * gamma_ref[...] + beta_ref[...]
    out_ref[...] = lax.dot_general(
        y, w_ref[...], (((1,), (1,)), ((), ())),
        preferred_element_type=jnp.float32) + b_ref[...]


def kernel(input_ids, word_emb, pos_emb, type_emb, ln_gamma, ln_beta, W, b):
    batch, seq = input_ids.shape
    ids_flat = input_ids.reshape(-1).astype(jnp.int32)
    gathered = _sc_gather(ids_flat, word_emb)

    pos = pos_emb[:seq]
    type0 = type_emb[0:1]
    gamma = ln_gamma.reshape(1, HID)
    beta = ln_beta.reshape(1, HID)
    bias = b.reshape(1, HID)

    blk = 512
    n_blk = (batch * seq) // blk
    pos_blocks = seq // blk
    out = pl.pallas_call(
        _tc_body,
        grid=(n_blk,),
        in_specs=[
            pl.BlockSpec((blk, HID), lambda i: (i, 0)),
            pl.BlockSpec((blk, HID), lambda i: (i % pos_blocks, 0)),
            pl.BlockSpec((1, HID), lambda i: (0, 0)),
            pl.BlockSpec((1, HID), lambda i: (0, 0)),
            pl.BlockSpec((1, HID), lambda i: (0, 0)),
            pl.BlockSpec((HID, HID), lambda i: (0, 0)),
            pl.BlockSpec((1, HID), lambda i: (0, 0)),
        ],
        out_specs=pl.BlockSpec((blk, HID), lambda i: (i, 0)),
        out_shape=jax.ShapeDtypeStruct((batch * seq, HID), jnp.float32),
    )(gathered, pos, type0, gamma, beta, W, bias)

    return out.reshape(batch, seq, HID)


# speedup vs baseline: 1.2440x; 1.2440x over previous
import functools

import jax
import jax.numpy as jnp
from jax import lax
from jax.experimental import pallas as pl
from jax.experimental.pallas import tpu as pltpu
from jax.experimental.pallas import tpu_sc as plsc

HID = 128
EPS = 1e-12

_SC_INFO = plsc.get_sparse_core_info()
_NC = _SC_INFO.num_cores
_NS = _SC_INFO.num_subcores
_NW = _NC * _NS

_CHUNK = 128


def _gather_body(ids_hbm, table_hbm, out_hbm,
                 idx0, idx1, rows0, rows1, gsem0, gsem1, wsem):
    wid = lax.axis_index("s") * _NC + lax.axis_index("c")
    base = wid * (2 * _CHUNK)
    pltpu.sync_copy(ids_hbm.at[pl.ds(base, _CHUNK)], idx0)
    g0 = pltpu.async_copy(table_hbm.at[idx0], rows0, gsem0)
    pltpu.sync_copy(ids_hbm.at[pl.ds(base + _CHUNK, _CHUNK)], idx1)
    g1 = pltpu.async_copy(table_hbm.at[idx1], rows1, gsem1)
    g0.wait()
    w0 = pltpu.async_copy(rows0, out_hbm.at[pl.ds(base, _CHUNK)], wsem)
    g1.wait()
    w1 = pltpu.async_copy(rows1, out_hbm.at[pl.ds(base + _CHUNK, _CHUNK)],
                          wsem)
    w0.wait()
    w1.wait()


def _sc_gather(ids_flat, word_emb):
    n_tokens = ids_flat.shape[0]
    assert n_tokens == _NW * 2 * _CHUNK
    mesh = plsc.VectorSubcoreMesh(core_axis_name="c", subcore_axis_name="s")
    k = functools.partial(
        pl.kernel,
        mesh=mesh,
        out_type=jax.ShapeDtypeStruct((n_tokens, HID), jnp.float32),
        scratch_types=[
            pltpu.VMEM((_CHUNK,), jnp.int32),
            pltpu.VMEM((_CHUNK,), jnp.int32),
            pltpu.VMEM((_CHUNK, HID), jnp.float32),
            pltpu.VMEM((_CHUNK, HID), jnp.float32),
            pltpu.SemaphoreType.DMA,
            pltpu.SemaphoreType.DMA,
            pltpu.SemaphoreType.DMA,
        ],
    )(_gather_body)
    return k(ids_flat, word_emb)


def _tc_body(x_ref, pos_ref, type_ref, gamma_ref, beta_ref, w_ref, b_ref,
             out_ref):
    x = x_ref[...] + pos_ref[...] + type_ref[...]
    mean = jnp.mean(x, axis=-1, keepdims=True)
    xc = x - mean
    var = jnp.mean(xc * xc, axis=-1, keepdims=True)
    normed = xc * lax.rsqrt(var + EPS)
    y = normed * gamma_ref[...] + beta_ref[...]
    out_ref[...] = lax.dot_general(
        y, w_ref[...], (((1,), (1,)), ((), ())),
        preferred_element_type=jnp.float32) + b_ref[...]


def kernel(input_ids, word_emb, pos_emb, type_emb, ln_gamma, ln_beta, W, b):
    batch, seq = input_ids.shape
    ids_flat = input_ids.reshape(-1).astype(jnp.int32)
    gathered = _sc_gather(ids_flat, word_emb)

    pos = pos_emb[:seq]
    type0 = type_emb[0:1]
    gamma = ln_gamma.reshape(1, HID)
    beta = ln_beta.reshape(1, HID)
    bias = b.reshape(1, HID)

    blk = 2048
    n_blk = (batch * seq) // blk
    pos_blocks = seq // blk
    out = pl.pallas_call(
        _tc_body,
        grid=(n_blk,),
        in_specs=[
            pl.BlockSpec((blk, HID), lambda i: (i, 0)),
            pl.BlockSpec((blk, HID), lambda i: (i % pos_blocks, 0)),
            pl.BlockSpec((1, HID), lambda i: (0, 0)),
            pl.BlockSpec((1, HID), lambda i: (0, 0)),
            pl.BlockSpec((1, HID), lambda i: (0, 0)),
            pl.BlockSpec((HID, HID), lambda i: (0, 0)),
            pl.BlockSpec((1, HID), lambda i: (0, 0)),
        ],
        out_specs=pl.BlockSpec((blk, HID), lambda i: (i, 0)),
        out_shape=jax.ShapeDtypeStruct((batch * seq, HID), jnp.float32),
    )(gathered, pos, type0, gamma, beta, W, bias)

    return out.reshape(batch, seq, HID)
